# Initial kernel scaffold; baseline (speedup 1.0000x reference)
import jax
import jax.numpy as jnp
from jax.experimental import pallas as pl


def kernel(input_ids, attention_mask, emb, W_r, b_r, W1, b1, W2, b2):
    raise NotImplementedError("write your pallas kernel here")


# speedup vs baseline: 34.4357x; 34.4357x over previous
import functools
import math

import jax
import jax.numpy as jnp
from jax.experimental import pallas as pl


def _tile_kernel(ids_ref, mask_ref, emb_ref, wr_ref, br_ref, w1_ref, b1_ref,
                 w2_ref, b2_ref, tv_ref, ti_ref, tw_ref, ts_ref):
    ids = ids_ref[...]
    maskf = mask_ref[...].astype(jnp.float32)
    V, D = emb_ref.shape
    TB = ids.shape[0]
    E = wr_ref.shape[1]

    pooled = jnp.zeros((TB, D), jnp.float32)
    for v in range(V):
        cnt = jnp.sum(jnp.where(ids == v, maskf, 0.0), axis=1)
        pooled = pooled + cnt[:, None] * emb_ref[v, :][None, :]
    denom = jnp.clip(jnp.sum(maskf, axis=1), 1e-9, None)
    pooled = pooled / denom[:, None]

    logits = jnp.dot(pooled, wr_ref[...], preferred_element_type=jnp.float32)
    logits = logits + br_ref[0, :][None, :]
    mx = jnp.max(logits, axis=1, keepdims=True)
    ex = jnp.exp(logits - mx)
    probs = ex / jnp.sum(ex, axis=1, keepdims=True)

    eio = jax.lax.broadcasted_iota(jnp.int32, (TB, E), 1)
    m1 = jnp.max(probs, axis=1, keepdims=True)
    i1 = jnp.min(jnp.where(probs == m1, eio, E), axis=1, keepdims=True)
    probs2 = jnp.where(eio == i1, -1.0, probs)
    m2 = jnp.max(probs2, axis=1, keepdims=True)
    i2 = jnp.min(jnp.where(probs2 == m2, eio, E), axis=1, keepdims=True)

    out_all = jnp.zeros((TB, E), jnp.float32)
    for e in range(E):
        h = jnp.dot(pooled, w1_ref[e], preferred_element_type=jnp.float32)
        h = jnp.maximum(h + b1_ref[e, :][None, :], 0.0)
        oe = jnp.sum(h * w2_ref[e, :][None, :], axis=1)
        out_all = out_all + oe[:, None] * (eio == e).astype(jnp.float32)
    out_all = out_all + b2_ref[0, :][None, :]

    t0 = jnp.sum(out_all * (eio == i1).astype(jnp.float32), axis=1, keepdims=True)
    t1 = jnp.sum(out_all * (eio == i2).astype(jnp.float32), axis=1, keepdims=True)

    wsum = jnp.clip(m1 + m2, 1e-9, None)
    tv_ref[:, 0:1] = m1
    tv_ref[:, 1:2] = m2
    ti_ref[:, 0:1] = i1.astype(jnp.float32)
    ti_ref[:, 1:2] = i2.astype(jnp.float32)
    tw_ref[:, 0:1] = m1 / wsum
    tw_ref[:, 1:2] = m2 / wsum
    ts_ref[:, 0:1] = t0
    ts_ref[:, 1:2] = t1


def _accept_kernel(v0r_ref, v1r_ref, e0r_ref, e1r_ref,
                   v0c_ref, v1c_ref, e0c_ref, e1c_ref,
                   w0_ref, w1_ref, t0_ref, t1_ref, out_ref,
                   *, capacity, chunk):
    nb = v0r_ref.shape[1]
    v0r = v0r_ref[...]
    v1r = v1r_ref[...]
    e0r = e0r_ref[...]
    e1r = e1r_ref[...]
    bi = jax.lax.broadcasted_iota(jnp.int32, (1, nb), 1)

    def body(c, carry):
        cnt0, cnt1 = carry
        s = c * chunk
        v0c = v0c_ref[pl.ds(s, chunk), :]
        v1c = v1c_ref[pl.ds(s, chunk), :]
        e0c = e0c_ref[pl.ds(s, chunk), :]
        e1c = e1c_ref[pl.ds(s, chunk), :]
        bj = jax.lax.broadcasted_iota(jnp.int32, (chunk, 1), 0) + s
        lt = bj < bi
        le = bj <= bi
        a00 = (e0c == e0r) & ((v0c > v0r) | ((v0c == v0r) & lt))
        a10 = (e1c == e0r) & ((v1c > v0r) | ((v1c == v0r) & lt))
        a01 = (e0c == e1r) & ((v0c > v1r) | ((v0c == v1r) & le))
        a11 = (e1c == e1r) & ((v1c > v1r) | ((v1c == v1r) & lt))
        cnt0 = cnt0 + jnp.sum(a00.astype(jnp.float32) + a10.astype(jnp.float32),
                              axis=0, keepdims=True)
        cnt1 = cnt1 + jnp.sum(a01.astype(jnp.float32) + a11.astype(jnp.float32),
                              axis=0, keepdims=True)
        return cnt0, cnt1

    z = jnp.zeros((1, nb), jnp.float32)
    cnt0, cnt1 = jax.lax.fori_loop(0, nb // chunk, body, (z, z))
    acc0 = (cnt0 < capacity).astype(jnp.float32)
    acc1 = (cnt1 < capacity).astype(jnp.float32)
    aw0 = w0_ref[...] * acc0
    aw1 = w1_ref[...] * acc1
    den = jnp.clip(aw0 + aw1, 1e-9, None)
    out_ref[...] = (aw0 * t0_ref[...] + aw1 * t1_ref[...]) / den


def kernel(input_ids, attention_mask, emb, W_r, b_r, W1, b1, W2, b2):
    B, L = input_ids.shape
    V, D = emb.shape
    E = W_r.shape[1]
    H = W1.shape[2]
    K = 2
    capacity = max(1, int(math.ceil(1.25 * B * K / E)))

    TB = 128
    grid = (B // TB,)
    full = lambda shape: pl.BlockSpec(shape, lambda i: tuple(0 for _ in shape))
    tv, ti, tw, ts = pl.pallas_call(
        _tile_kernel,
        grid=grid,
        in_specs=[
            pl.BlockSpec((TB, L), lambda i: (i, 0)),
            pl.BlockSpec((TB, L), lambda i: (i, 0)),
            full((V, D)),
            full((D, E)),
            full((1, E)),
            full((E, D, H)),
            full((E, H)),
            full((E, H)),
            full((1, E)),
        ],
        out_specs=[pl.BlockSpec((TB, 2), lambda i: (i, 0))] * 4,
        out_shape=[jax.ShapeDtypeStruct((B, 2), jnp.float32)] * 4,
    )(input_ids, attention_mask, emb, W_r, b_r.reshape(1, E),
      W1, b1, W2, b2.reshape(1, E))

    row = lambda a: a.reshape(1, B)
    scores = pl.pallas_call(
        functools.partial(_accept_kernel, capacity=capacity, chunk=256),
        in_specs=[pl.BlockSpec((1, B), lambda: (0, 0))] * 4
                 + [pl.BlockSpec((B, 1), lambda: (0, 0))] * 4
                 + [pl.BlockSpec((1, B), lambda: (0, 0))] * 4,
        out_specs=pl.BlockSpec((1, B), lambda: (0, 0)),
        out_shape=jax.ShapeDtypeStruct((1, B), jnp.float32),
    )(row(tv[:, 0]), row(tv[:, 1]), row(ti[:, 0]), row(ti[:, 1]),
      tv[:, 0:1], tv[:, 1:2], ti[:, 0:1], ti[:, 1:2],
      row(tw[:, 0]), row(tw[:, 1]), row(ts[:, 0]), row(ts[:, 1]))
    return scores.reshape(B)
